# baseline (speedup 1.0000x reference)
import functools

import jax
import jax.numpy as jnp
from jax import lax
from jax.experimental import pallas as pl
from jax.experimental.pallas import tpu as pltpu
from jax.experimental.pallas import tpu_sc as plsc

_LOG2 = 0.6931471805599453
_LOG2E = 1.4426950408889634


def _ssp(x):
    return jnp.maximum(x, 0.0) + jnp.log1p(jnp.exp(-jnp.abs(x))) - _LOG2


def _sp_fast(x):
    return jnp.maximum(x, 0.0) + jnp.log1p(jnp.exp(jnp.minimum(x, -x)))


def _node_body(x_ref, w_ref, b_ref, o_ref):
    o_ref[...] = (
        jnp.dot(x_ref[...], w_ref[...], preferred_element_type=jnp.float32)
        + b_ref[...]
    )


def _node_proj(x, W, b):
    N, D = x.shape
    H = W.shape[1]
    BN = 1000
    return pl.pallas_call(
        _node_body,
        grid=(N // BN,),
        in_specs=[
            pl.BlockSpec((BN, D), lambda i: (i, 0)),
            pl.BlockSpec((D, H), lambda i: (0, 0)),
            pl.BlockSpec((1, H), lambda i: (0, 0)),
        ],
        out_specs=pl.BlockSpec((BN, H), lambda i: (i, 0)),
        out_shape=jax.ShapeDtypeStruct((N, H), jnp.float32),
    )(x, W, b.reshape(1, H))


def _edge_body(x_ref, w1_ref, b1_ref, w2_ref, b2_ref, o_ref):
    h1 = _ssp(
        jnp.dot(x_ref[...], w1_ref[...], preferred_element_type=jnp.float32)
        + b1_ref[...]
    )
    o_ref[...] = _ssp(
        jnp.dot(h1, w2_ref[...], preferred_element_type=jnp.float32)
        + b2_ref[...]
    )


def _edge_mlp(e, W1, b1, W2, b2):
    E, DE = e.shape
    H = W1.shape[1]
    BE = 8000
    b2_eff = b2
    return pl.pallas_call(
        _edge_body,
        grid=(E // BE,),
        in_specs=[
            pl.BlockSpec((BE, DE), lambda i: (i, 0)),
            pl.BlockSpec((DE, H), lambda i: (0, 0)),
            pl.BlockSpec((1, H), lambda i: (0, 0)),
            pl.BlockSpec((H, H), lambda i: (0, 0)),
            pl.BlockSpec((1, H), lambda i: (0, 0)),
        ],
        out_specs=pl.BlockSpec((BE, H), lambda i: (i, 0)),
        out_shape=jax.ShapeDtypeStruct((E, H), jnp.float32),
    )(e, W1, b1.reshape(1, H), W2, b2_eff.reshape(1, H))


def _out_body(pa_ref, pb_ref, wcf_ref, bcf_ref, wout_ref, bout_ref, o_ref):
    agg = pa_ref[0] + pa_ref[1] + pb_ref[0] + pb_ref[1]
    h = _ssp(
        jnp.dot(agg, wcf_ref[...], preferred_element_type=jnp.float32)
        + bcf_ref[...]
    )
    o_ref[...] = (
        jnp.dot(h, wout_ref[...], preferred_element_type=jnp.float32)
        + bout_ref[...]
    )


def _out_proj(pa, pb, W_cf, b_cf, W_out, b_out):
    _, N, H = pa.shape
    D = W_cf.shape[1]
    BN = 1000
    return pl.pallas_call(
        _out_body,
        grid=(N // BN,),
        in_specs=[
            pl.BlockSpec((2, BN, H), lambda i: (0, i, 0)),
            pl.BlockSpec((2, BN, H), lambda i: (0, i, 0)),
            pl.BlockSpec((H, D), lambda i: (0, 0)),
            pl.BlockSpec((1, D), lambda i: (0, 0)),
            pl.BlockSpec((D, D), lambda i: (0, 0)),
            pl.BlockSpec((1, D), lambda i: (0, 0)),
        ],
        out_specs=pl.BlockSpec((BN, D), lambda i: (i, 0)),
        out_shape=jax.ShapeDtypeStruct((N, D), jnp.float32),
    )(pa, pb, W_cf, b_cf.reshape(1, D), W_out, b_out.reshape(1, D))


_NC = 2
_NS = 16
_C = 96


@functools.lru_cache(maxsize=None)
def _make_sc_gather_scatter(N, E, H):
    NW = _NC * _NS
    assert E % NW == 0
    EPW = E // NW
    nchunk = EPW // _C
    tail = EPW % _C
    assert tail % 8 == 0 and tail > 0 and nchunk % 2 == 0
    npairs = nchunk // 2
    nflush = 10
    rows_pt = N // nflush
    assert N % nflush == 0 and rows_pt % 8 == 0
    lanes = 16
    assert H % lanes == 0

    mesh = plsc.VectorSubcoreMesh(
        core_axis_name="c", subcore_axis_name="s",
        num_cores=_NC, num_subcores=_NS,
    )

    @functools.partial(
        pl.kernel,
        out_type=jax.ShapeDtypeStruct((_NC, N, H), jnp.float32),
        mesh=mesh,
        scratch_types=[
            pltpu.VMEM((_C,), jnp.int32),
            pltpu.VMEM((_C,), jnp.int32),
            pltpu.VMEM((_C, H), jnp.float32),
            pltpu.VMEM((_C, H), jnp.float32),
            pltpu.VMEM((_C,), jnp.int32),
            pltpu.VMEM((_C,), jnp.int32),
            pltpu.VMEM((_C, H), jnp.float32),
            pltpu.VMEM((_C, H), jnp.float32),
            pltpu.VMEM((tail,), jnp.int32),
            pltpu.VMEM((tail,), jnp.int32),
            pltpu.VMEM_SHARED((N, H), jnp.float32),
            pltpu.SemaphoreType.DMA,
            pltpu.SemaphoreType.DMA,
            pltpu.SemaphoreType.DMA,
            pltpu.SemaphoreType.DMA,
            pltpu.SemaphoreType.DMA,
            pltpu.SemaphoreType.DMA,
        ],
    )
    def sc_kernel(hv_hbm, he_hbm, src_hbm, dst_hbm, out_hbm,
                  src0, dst0, rows0, he0,
                  src1, dst1, rows1, he1,
                  src_t, dst_t,
                  agg, sg0, sh0, ss0, sg1, sh1, ss1):
        cid = lax.axis_index("c")
        sid = lax.axis_index("s")
        wid = cid * _NS + sid
        base = wid * EPW
        bufs = ((src0, dst0, rows0, he0, sg0, sh0, ss0),
                (src1, dst1, rows1, he1, sg1, sh1, ss1))

        @pl.when(sid < nflush)
        def _zero():
            def _zrow(rr, carry):
                for j in range(H // lanes):
                    rows0[rr, pl.ds(j * lanes, lanes)] = jnp.zeros(
                        (lanes,), jnp.float32)
                return carry
            lax.fori_loop(0, _C, _zrow, 0)
            done = 0
            while done < rows_pt:
                n = min(_C, rows_pt - done)
                assert n % 8 == 0
                pltpu.sync_copy(
                    rows0.at[pl.ds(0, n)],
                    agg.at[pl.ds(sid * rows_pt + done, n)])
                done += n
        plsc.subcore_barrier()

        def _issue(off, b):
            s_v, d_v, r_v, h_v, sg, sh, _ = bufs[b]
            pltpu.sync_copy(src_hbm.at[pl.ds(off, _C)], s_v)
            pltpu.sync_copy(dst_hbm.at[pl.ds(off, _C)], d_v)
            pltpu.async_copy(hv_hbm.at[s_v], r_v, sg)
            pltpu.async_copy(he_hbm.at[pl.ds(off, _C)], h_v, sh)

        def _wait_in(b):
            s_v, _, r_v, h_v, sg, sh, _ = bufs[b]
            pltpu.make_async_copy(hv_hbm.at[s_v], r_v, sg).wait()
            pltpu.make_async_copy(he_hbm.at[pl.ds(0, _C)], h_v, sh).wait()

        def _mul(r_v, h_v, n):
            @plsc.parallel_loop(0, n, unroll=2)
            def _mrow(rr):
                for j in range(H // lanes):
                    sl = pl.ds(j * lanes, lanes)
                    r_v[rr, sl] = r_v[rr, sl] * h_v[rr, sl]

        def _scatter(b):
            _, d_v, r_v, _, _, _, ss = bufs[b]
            pltpu.async_copy(r_v, agg.at[d_v], ss, add=True)

        def _wait_scatter(b):
            _, d_v, r_v, _, _, _, ss = bufs[b]
            pltpu.make_async_copy(r_v, agg.at[d_v], ss).wait()

        _issue(base, 0)

        def _pair(p, carry):
            off0 = base + (2 * p) * _C
            _wait_in(0)
            _issue(off0 + _C, 1)
            _mul(rows0, he0, _C)
            _scatter(0)
            _wait_in(1)
            _wait_scatter(0)

            @pl.when(2 * p + 2 < nchunk)
            def _():
                _issue(off0 + 2 * _C, 0)
            _mul(rows1, he1, _C)
            _scatter(1)
            _wait_scatter(1)
            return carry
        lax.fori_loop(0, npairs, _pair, 0)

        toff = base + nchunk * _C
        pltpu.sync_copy(src_hbm.at[pl.ds(toff, tail)], src_t)
        pltpu.sync_copy(dst_hbm.at[pl.ds(toff, tail)], dst_t)
        pltpu.async_copy(
            hv_hbm.at[src_t], rows0.at[pl.ds(0, tail)], sg0).wait()
        pltpu.sync_copy(he_hbm.at[pl.ds(toff, tail)], he0.at[pl.ds(0, tail)])
        _mul(rows0, he0, tail)
        pltpu.sync_copy(rows0.at[pl.ds(0, tail)], agg.at[dst_t], add=True)

        plsc.subcore_barrier()

        @pl.when(sid < nflush)
        def _flush():
            pltpu.sync_copy(
                agg.at[pl.ds(sid * rows_pt, rows_pt)],
                out_hbm.at[cid, pl.ds(sid * rows_pt, rows_pt)],
            )

    return sc_kernel


def kernel(node_feats, edge_feats, edge_index,
           W_node, b_node, W_e1, b_e1, W_e2, b_e2,
           W_cf, b_cf, W_out, b_out):
    N, D = node_feats.shape
    E = edge_feats.shape[0]
    H = W_node.shape[1]
    src = edge_index[0]
    dst = edge_index[1]
    hv = _node_proj(node_feats, W_node, b_node)
    Eh = E // 2
    sc = _make_sc_gather_scatter(N, Eh, H)
    he_a = _edge_mlp(edge_feats[:Eh], W_e1, b_e1, W_e2, b_e2)
    pa = sc(hv, he_a, src[:Eh], dst[:Eh])
    he_b = _edge_mlp(edge_feats[Eh:], W_e1, b_e1, W_e2, b_e2)
    pb = sc(hv, he_b, src[Eh:], dst[Eh:])
    return _out_proj(pa, pb, W_cf, b_cf, W_out, b_out)


# speedup vs baseline: 1.0047x; 1.0047x over previous
import functools

import jax
import jax.numpy as jnp
from jax import lax
from jax.experimental import pallas as pl
from jax.experimental.pallas import tpu as pltpu
from jax.experimental.pallas import tpu_sc as plsc

_LOG2 = 0.6931471805599453
_LOG2E = 1.4426950408889634


def _ssp(x):
    return jnp.maximum(x, 0.0) + jnp.log1p(jnp.exp(-jnp.abs(x))) - _LOG2


def _sp_fast(x):
    return jnp.maximum(x, 0.0) + jnp.log1p(jnp.exp(jnp.minimum(x, -x)))


def _node_body(x_ref, w_ref, b_ref, o_ref):
    o_ref[...] = (
        jnp.dot(x_ref[...], w_ref[...], preferred_element_type=jnp.float32)
        + b_ref[...]
    )


def _node_proj(x, W, b):
    N, D = x.shape
    H = W.shape[1]
    BN = 1000
    return pl.pallas_call(
        _node_body,
        grid=(N // BN,),
        in_specs=[
            pl.BlockSpec((BN, D), lambda i: (i, 0)),
            pl.BlockSpec((D, H), lambda i: (0, 0)),
            pl.BlockSpec((1, H), lambda i: (0, 0)),
        ],
        out_specs=pl.BlockSpec((BN, H), lambda i: (i, 0)),
        out_shape=jax.ShapeDtypeStruct((N, H), jnp.float32),
    )(x, W, b.reshape(1, H))


def _edge_body(x_ref, w1_ref, b1_ref, w2_ref, b2_ref, o_ref):
    h1 = _ssp(
        jnp.dot(x_ref[...], w1_ref[...], preferred_element_type=jnp.float32)
        + b1_ref[...]
    )
    o_ref[...] = _ssp(
        jnp.dot(h1, w2_ref[...], preferred_element_type=jnp.float32)
        + b2_ref[...]
    )


def _edge_mlp(e, W1, b1, W2, b2):
    E, DE = e.shape
    H = W1.shape[1]
    BE = 4000
    b2_eff = b2
    return pl.pallas_call(
        _edge_body,
        grid=(E // BE,),
        in_specs=[
            pl.BlockSpec((BE, DE), lambda i: (i, 0)),
            pl.BlockSpec((DE, H), lambda i: (0, 0)),
            pl.BlockSpec((1, H), lambda i: (0, 0)),
            pl.BlockSpec((H, H), lambda i: (0, 0)),
            pl.BlockSpec((1, H), lambda i: (0, 0)),
        ],
        out_specs=pl.BlockSpec((BE, H), lambda i: (i, 0)),
        out_shape=jax.ShapeDtypeStruct((E, H), jnp.float32),
    )(e, W1, b1.reshape(1, H), W2, b2_eff.reshape(1, H))


def _out_body(pa_ref, pb_ref, wcf_ref, bcf_ref, wout_ref, bout_ref, o_ref):
    agg = pa_ref[0] + pa_ref[1] + pb_ref[0] + pb_ref[1]
    h = _ssp(
        jnp.dot(agg, wcf_ref[...], preferred_element_type=jnp.float32)
        + bcf_ref[...]
    )
    o_ref[...] = (
        jnp.dot(h, wout_ref[...], preferred_element_type=jnp.float32)
        + bout_ref[...]
    )


def _out_proj(pa, pb, W_cf, b_cf, W_out, b_out):
    _, N, H = pa.shape
    D = W_cf.shape[1]
    BN = 1000
    return pl.pallas_call(
        _out_body,
        grid=(N // BN,),
        in_specs=[
            pl.BlockSpec((2, BN, H), lambda i: (0, i, 0)),
            pl.BlockSpec((2, BN, H), lambda i: (0, i, 0)),
            pl.BlockSpec((H, D), lambda i: (0, 0)),
            pl.BlockSpec((1, D), lambda i: (0, 0)),
            pl.BlockSpec((D, D), lambda i: (0, 0)),
            pl.BlockSpec((1, D), lambda i: (0, 0)),
        ],
        out_specs=pl.BlockSpec((BN, D), lambda i: (i, 0)),
        out_shape=jax.ShapeDtypeStruct((N, D), jnp.float32),
    )(pa, pb, W_cf, b_cf.reshape(1, D), W_out, b_out.reshape(1, D))


_NC = 2
_NS = 16
_C = 96


@functools.lru_cache(maxsize=None)
def _make_sc_gather_scatter(N, E, H):
    NW = _NC * _NS
    assert E % NW == 0
    EPW = E // NW
    nchunk = EPW // _C
    tail = EPW % _C
    assert tail % 8 == 0 and tail > 0 and nchunk % 2 == 0
    npairs = nchunk // 2
    nflush = 10
    rows_pt = N // nflush
    assert N % nflush == 0 and rows_pt % 8 == 0
    lanes = 16
    assert H % lanes == 0

    mesh = plsc.VectorSubcoreMesh(
        core_axis_name="c", subcore_axis_name="s",
        num_cores=_NC, num_subcores=_NS,
    )

    @functools.partial(
        pl.kernel,
        out_type=jax.ShapeDtypeStruct((_NC, N, H), jnp.float32),
        mesh=mesh,
        scratch_types=[
            pltpu.VMEM((_C,), jnp.int32),
            pltpu.VMEM((_C,), jnp.int32),
            pltpu.VMEM((_C, H), jnp.float32),
            pltpu.VMEM((_C, H), jnp.float32),
            pltpu.VMEM((_C,), jnp.int32),
            pltpu.VMEM((_C,), jnp.int32),
            pltpu.VMEM((_C, H), jnp.float32),
            pltpu.VMEM((_C, H), jnp.float32),
            pltpu.VMEM((tail,), jnp.int32),
            pltpu.VMEM((tail,), jnp.int32),
            pltpu.VMEM_SHARED((N, H), jnp.float32),
            pltpu.SemaphoreType.DMA,
            pltpu.SemaphoreType.DMA,
            pltpu.SemaphoreType.DMA,
            pltpu.SemaphoreType.DMA,
            pltpu.SemaphoreType.DMA,
            pltpu.SemaphoreType.DMA,
        ],
    )
    def sc_kernel(hv_hbm, he_hbm, src_hbm, dst_hbm, out_hbm,
                  src0, dst0, rows0, he0,
                  src1, dst1, rows1, he1,
                  src_t, dst_t,
                  agg, sg0, sh0, ss0, sg1, sh1, ss1):
        cid = lax.axis_index("c")
        sid = lax.axis_index("s")
        wid = cid * _NS + sid
        base = wid * EPW
        bufs = ((src0, dst0, rows0, he0, sg0, sh0, ss0),
                (src1, dst1, rows1, he1, sg1, sh1, ss1))

        @pl.when(sid < nflush)
        def _zero():
            def _zrow(rr, carry):
                for j in range(H // lanes):
                    rows0[rr, pl.ds(j * lanes, lanes)] = jnp.zeros(
                        (lanes,), jnp.float32)
                return carry
            lax.fori_loop(0, _C, _zrow, 0)
            done = 0
            while done < rows_pt:
                n = min(_C, rows_pt - done)
                assert n % 8 == 0
                pltpu.sync_copy(
                    rows0.at[pl.ds(0, n)],
                    agg.at[pl.ds(sid * rows_pt + done, n)])
                done += n
        plsc.subcore_barrier()

        def _issue(off, b):
            s_v, d_v, r_v, h_v, sg, sh, _ = bufs[b]
            pltpu.sync_copy(src_hbm.at[pl.ds(off, _C)], s_v)
            pltpu.sync_copy(dst_hbm.at[pl.ds(off, _C)], d_v)
            pltpu.async_copy(hv_hbm.at[s_v], r_v, sg)
            pltpu.async_copy(he_hbm.at[pl.ds(off, _C)], h_v, sh)

        def _wait_in(b):
            s_v, _, r_v, h_v, sg, sh, _ = bufs[b]
            pltpu.make_async_copy(hv_hbm.at[s_v], r_v, sg).wait()
            pltpu.make_async_copy(he_hbm.at[pl.ds(0, _C)], h_v, sh).wait()

        def _mul(r_v, h_v, n):
            @plsc.parallel_loop(0, n, unroll=2)
            def _mrow(rr):
                for j in range(H // lanes):
                    sl = pl.ds(j * lanes, lanes)
                    r_v[rr, sl] = r_v[rr, sl] * h_v[rr, sl]

        def _scatter(b):
            _, d_v, r_v, _, _, _, ss = bufs[b]
            pltpu.async_copy(r_v, agg.at[d_v], ss, add=True)

        def _wait_scatter(b):
            _, d_v, r_v, _, _, _, ss = bufs[b]
            pltpu.make_async_copy(r_v, agg.at[d_v], ss).wait()

        _issue(base, 0)

        def _pair(p, carry):
            off0 = base + (2 * p) * _C
            _wait_in(0)
            _issue(off0 + _C, 1)
            _mul(rows0, he0, _C)
            _scatter(0)
            _wait_in(1)
            _wait_scatter(0)

            @pl.when(2 * p + 2 < nchunk)
            def _():
                _issue(off0 + 2 * _C, 0)
            _mul(rows1, he1, _C)
            _scatter(1)
            _wait_scatter(1)
            return carry
        lax.fori_loop(0, npairs, _pair, 0)

        toff = base + nchunk * _C
        pltpu.sync_copy(src_hbm.at[pl.ds(toff, tail)], src_t)
        pltpu.sync_copy(dst_hbm.at[pl.ds(toff, tail)], dst_t)
        pltpu.async_copy(
            hv_hbm.at[src_t], rows0.at[pl.ds(0, tail)], sg0).wait()
        pltpu.sync_copy(he_hbm.at[pl.ds(toff, tail)], he0.at[pl.ds(0, tail)])
        _mul(rows0, he0, tail)
        pltpu.sync_copy(rows0.at[pl.ds(0, tail)], agg.at[dst_t], add=True)

        plsc.subcore_barrier()

        @pl.when(sid < nflush)
        def _flush():
            pltpu.sync_copy(
                agg.at[pl.ds(sid * rows_pt, rows_pt)],
                out_hbm.at[cid, pl.ds(sid * rows_pt, rows_pt)],
            )

    return sc_kernel


def kernel(node_feats, edge_feats, edge_index,
           W_node, b_node, W_e1, b_e1, W_e2, b_e2,
           W_cf, b_cf, W_out, b_out):
    N, D = node_feats.shape
    E = edge_feats.shape[0]
    H = W_node.shape[1]
    src = edge_index[0]
    dst = edge_index[1]
    hv = _node_proj(node_feats, W_node, b_node)
    Eh = E // 2
    sc = _make_sc_gather_scatter(N, Eh, H)
    he_a = _edge_mlp(edge_feats[:Eh], W_e1, b_e1, W_e2, b_e2)
    pa = sc(hv, he_a, src[:Eh], dst[:Eh])
    he_b = _edge_mlp(edge_feats[Eh:], W_e1, b_e1, W_e2, b_e2)
    pb = sc(hv, he_b, src[Eh:], dst[Eh:])
    return _out_proj(pa, pb, W_cf, b_cf, W_out, b_out)


# speedup vs baseline: 1.0063x; 1.0016x over previous
import functools

import jax
import jax.numpy as jnp
from jax import lax
from jax.experimental import pallas as pl
from jax.experimental.pallas import tpu as pltpu
from jax.experimental.pallas import tpu_sc as plsc

_LOG2 = 0.6931471805599453


def _ssp(x):
    return jnp.maximum(x, 0.0) + jnp.log1p(jnp.exp(-jnp.abs(x))) - _LOG2


def _node_body(x_ref, w_ref, b_ref, o_ref):
    o_ref[...] = (
        jnp.dot(x_ref[...], w_ref[...], preferred_element_type=jnp.float32)
        + b_ref[...]
    )


def _node_proj(x, W, b):
    N, D = x.shape
    H = W.shape[1]
    BN = 1000
    return pl.pallas_call(
        _node_body,
        grid=(N // BN,),
        in_specs=[
            pl.BlockSpec((BN, D), lambda i: (i, 0)),
            pl.BlockSpec((D, H), lambda i: (0, 0)),
            pl.BlockSpec((1, H), lambda i: (0, 0)),
        ],
        out_specs=pl.BlockSpec((BN, H), lambda i: (i, 0)),
        out_shape=jax.ShapeDtypeStruct((N, H), jnp.float32),
    )(x, W, b.reshape(1, H))


def _edge_body(x_ref, w1_ref, b1_ref, w2_ref, b2_ref, o_ref):
    h1 = _ssp(
        jnp.dot(x_ref[...], w1_ref[...], preferred_element_type=jnp.float32)
        + b1_ref[...]
    )
    o_ref[...] = _ssp(
        jnp.dot(h1, w2_ref[...], preferred_element_type=jnp.float32)
        + b2_ref[...]
    )


def _edge_mlp(e, W1, b1, W2, b2):
    E, DE = e.shape
    H = W1.shape[1]
    BE = 4000
    return pl.pallas_call(
        _edge_body,
        grid=(E // BE,),
        in_specs=[
            pl.BlockSpec((BE, DE), lambda i: (i, 0)),
            pl.BlockSpec((DE, H), lambda i: (0, 0)),
            pl.BlockSpec((1, H), lambda i: (0, 0)),
            pl.BlockSpec((H, H), lambda i: (0, 0)),
            pl.BlockSpec((1, H), lambda i: (0, 0)),
        ],
        out_specs=pl.BlockSpec((BE, H), lambda i: (i, 0)),
        out_shape=jax.ShapeDtypeStruct((E, H), jnp.float32),
    )(e, W1, b1.reshape(1, H), W2, b2.reshape(1, H))


def _out_body(pa_ref, pb_ref, wcf_ref, bcf_ref, wout_ref, bout_ref, o_ref):
    agg = pa_ref[0] + pa_ref[1] + pb_ref[0] + pb_ref[1]
    h = _ssp(
        jnp.dot(agg, wcf_ref[...], preferred_element_type=jnp.float32)
        + bcf_ref[...]
    )
    o_ref[...] = (
        jnp.dot(h, wout_ref[...], preferred_element_type=jnp.float32)
        + bout_ref[...]
    )


def _out_proj(pa, pb, W_cf, b_cf, W_out, b_out):
    _, N, H = pa.shape
    D = W_cf.shape[1]
    BN = 1000
    return pl.pallas_call(
        _out_body,
        grid=(N // BN,),
        in_specs=[
            pl.BlockSpec((2, BN, H), lambda i: (0, i, 0)),
            pl.BlockSpec((2, BN, H), lambda i: (0, i, 0)),
            pl.BlockSpec((H, D), lambda i: (0, 0)),
            pl.BlockSpec((1, D), lambda i: (0, 0)),
            pl.BlockSpec((D, D), lambda i: (0, 0)),
            pl.BlockSpec((1, D), lambda i: (0, 0)),
        ],
        out_specs=pl.BlockSpec((BN, D), lambda i: (i, 0)),
        out_shape=jax.ShapeDtypeStruct((N, D), jnp.float32),
    )(pa, pb, W_cf, b_cf.reshape(1, D), W_out, b_out.reshape(1, D))


_NC = 2
_NS = 16
_C = 96


@functools.lru_cache(maxsize=None)
def _make_sc_gather_scatter(N, E, H):
    NW = _NC * _NS
    assert E % NW == 0
    EPW = E // NW
    nchunk = EPW // _C
    tail = EPW % _C
    assert tail % 8 == 0 and tail > 0 and nchunk % 2 == 0
    npairs = nchunk // 2
    nflush = 10
    rows_pt = N // nflush
    assert N % nflush == 0 and rows_pt % 8 == 0
    lanes = 16
    assert H % lanes == 0

    mesh = plsc.VectorSubcoreMesh(
        core_axis_name="c", subcore_axis_name="s",
        num_cores=_NC, num_subcores=_NS,
    )

    @functools.partial(
        pl.kernel,
        out_type=jax.ShapeDtypeStruct((_NC, N, H), jnp.float32),
        mesh=mesh,
        scratch_types=[
            pltpu.VMEM((_C,), jnp.int32),
            pltpu.VMEM((_C,), jnp.int32),
            pltpu.VMEM((_C, H), jnp.float32),
            pltpu.VMEM((_C, H), jnp.float32),
            pltpu.VMEM((_C,), jnp.int32),
            pltpu.VMEM((_C,), jnp.int32),
            pltpu.VMEM((_C, H), jnp.float32),
            pltpu.VMEM((_C, H), jnp.float32),
            pltpu.VMEM((tail,), jnp.int32),
            pltpu.VMEM((tail,), jnp.int32),
            pltpu.VMEM_SHARED((N, H), jnp.float32),
            pltpu.SemaphoreType.DMA,
            pltpu.SemaphoreType.DMA,
            pltpu.SemaphoreType.DMA,
            pltpu.SemaphoreType.DMA,
            pltpu.SemaphoreType.DMA,
            pltpu.SemaphoreType.DMA,
        ],
    )
    def sc_kernel(hv_hbm, he_hbm, src_hbm, dst_hbm, out_hbm,
                  src0, dst0, rows0, he0,
                  src1, dst1, rows1, he1,
                  src_t, dst_t,
                  agg, sg0, sh0, ss0, sg1, sh1, ss1):
        cid = lax.axis_index("c")
        sid = lax.axis_index("s")
        wid = cid * _NS + sid
        base = wid * EPW
        bufs = ((src0, dst0, rows0, he0, sg0, sh0, ss0),
                (src1, dst1, rows1, he1, sg1, sh1, ss1))

        @pl.when(sid < nflush)
        def _zero():
            def _zrow(rr, carry):
                for j in range(H // lanes):
                    rows0[rr, pl.ds(j * lanes, lanes)] = jnp.zeros(
                        (lanes,), jnp.float32)
                return carry
            lax.fori_loop(0, _C, _zrow, 0)
            done = 0
            while done < rows_pt:
                n = min(_C, rows_pt - done)
                assert n % 8 == 0
                pltpu.sync_copy(
                    rows0.at[pl.ds(0, n)],
                    agg.at[pl.ds(sid * rows_pt + done, n)])
                done += n
        plsc.subcore_barrier()

        def _issue(off, b):
            s_v, d_v, r_v, h_v, sg, sh, _ = bufs[b]
            pltpu.sync_copy(src_hbm.at[pl.ds(off, _C)], s_v)
            pltpu.sync_copy(dst_hbm.at[pl.ds(off, _C)], d_v)
            pltpu.async_copy(hv_hbm.at[s_v], r_v, sg)
            pltpu.async_copy(he_hbm.at[pl.ds(off, _C)], h_v, sh)

        def _wait_in(b):
            s_v, _, r_v, h_v, sg, sh, _ = bufs[b]
            pltpu.make_async_copy(hv_hbm.at[s_v], r_v, sg).wait()
            pltpu.make_async_copy(he_hbm.at[pl.ds(0, _C)], h_v, sh).wait()

        def _mul(r_v, h_v, n):
            @plsc.parallel_loop(0, n, unroll=2)
            def _mrow(rr):
                for j in range(H // lanes):
                    sl = pl.ds(j * lanes, lanes)
                    r_v[rr, sl] = r_v[rr, sl] * h_v[rr, sl]

        def _scatter(b):
            _, d_v, r_v, _, _, _, ss = bufs[b]
            pltpu.async_copy(r_v, agg.at[d_v], ss, add=True)

        def _wait_scatter(b):
            _, d_v, r_v, _, _, _, ss = bufs[b]
            pltpu.make_async_copy(r_v, agg.at[d_v], ss).wait()

        _issue(base, 0)

        def _pair(p, carry):
            off0 = base + (2 * p) * _C
            _wait_in(0)
            _issue(off0 + _C, 1)
            _mul(rows0, he0, _C)
            _scatter(0)
            _wait_in(1)
            _wait_scatter(0)

            @pl.when(2 * p + 2 < nchunk)
            def _():
                _issue(off0 + 2 * _C, 0)
            _mul(rows1, he1, _C)
            _scatter(1)
            _wait_scatter(1)
            return carry
        lax.fori_loop(0, npairs, _pair, 0)

        toff = base + nchunk * _C
        pltpu.sync_copy(src_hbm.at[pl.ds(toff, tail)], src_t)
        pltpu.sync_copy(dst_hbm.at[pl.ds(toff, tail)], dst_t)
        pltpu.async_copy(
            hv_hbm.at[src_t], rows0.at[pl.ds(0, tail)], sg0).wait()
        pltpu.sync_copy(he_hbm.at[pl.ds(toff, tail)], he0.at[pl.ds(0, tail)])
        _mul(rows0, he0, tail)
        pltpu.sync_copy(rows0.at[pl.ds(0, tail)], agg.at[dst_t], add=True)

        plsc.subcore_barrier()

        @pl.when(sid < nflush)
        def _flush():
            pltpu.sync_copy(
                agg.at[pl.ds(sid * rows_pt, rows_pt)],
                out_hbm.at[cid, pl.ds(sid * rows_pt, rows_pt)],
            )

    return sc_kernel


def kernel(node_feats, edge_feats, edge_index,
           W_node, b_node, W_e1, b_e1, W_e2, b_e2,
           W_cf, b_cf, W_out, b_out):
    N, D = node_feats.shape
    E = edge_feats.shape[0]
    H = W_node.shape[1]
    src = edge_index[0]
    dst = edge_index[1]
    hv = _node_proj(node_feats, W_node, b_node)
    Eh = E // 2
    sc = _make_sc_gather_scatter(N, Eh, H)
    he_a = _edge_mlp(edge_feats[:Eh], W_e1, b_e1, W_e2, b_e2)
    pa = sc(hv, he_a, src[:Eh], dst[:Eh])
    he_b = _edge_mlp(edge_feats[Eh:], W_e1, b_e1, W_e2, b_e2)
    pb = sc(hv, he_b, src[Eh:], dst[Eh:])
    return _out_proj(pa, pb, W_cf, b_cf, W_out, b_out)
